# Initial kernel scaffold; baseline (speedup 1.0000x reference)
import jax
import jax.numpy as jnp
from jax.experimental import pallas as pl


def kernel(vocab_dist, attn_dist, context, state, emb, src_ids, vocab_size, W_c, W_s, W_y, b):
    raise NotImplementedError("write your pallas kernel here")


# speedup vs baseline: 1.5706x; 1.5706x over previous
import functools

import jax
import jax.numpy as jnp
from jax import lax
from jax.experimental import pallas as pl
from jax.experimental.pallas import tpu as pltpu
from jax.experimental.pallas import tpu_sc as plsc

B = 128
V = 50000
S = 512
FEAT = 2560
NC = 2
NS = 16
L = 16
NW = NC * NS
ROWS_PER_W = B // NW

_MESH = plsc.VectorSubcoreMesh(
    core_axis_name="c", subcore_axis_name="s", num_cores=NC, num_subcores=NS
)


@functools.partial(
    pl.kernel,
    out_type=jax.ShapeDtypeStruct((B, V), jnp.float32),
    mesh=_MESH,
    compiler_params=pltpu.CompilerParams(needs_layout_passes=False),
    scratch_types=[
        pltpu.VMEM((V,), jnp.float32),
        pltpu.VMEM((V,), jnp.float32),
        pltpu.VMEM((FEAT,), jnp.float32),
        pltpu.VMEM((S,), jnp.float32),
        pltpu.VMEM((S,), jnp.int32),
        pltpu.VMEM((FEAT,), jnp.float32),
        pltpu.VMEM((L,), jnp.float32),
        pltpu.VMEM((L,), jnp.int32),
        pltpu.VMEM((L,), jnp.float32),
    ],
)
def _pg_kernel(vocab_hbm, feat_hbm, attn_hbm, src_hbm, w_hbm, b_hbm, vsz_hbm,
               out_hbm, row_a, row_b, feat_v, attn_v, src_v, w_v, b_v, vsz_v,
               red_v):
    wid = lax.axis_index("s") * NC + lax.axis_index("c")
    base = wid * ROWS_PER_W
    pltpu.sync_copy(w_hbm, w_v)
    pltpu.sync_copy(b_hbm, b_v)
    pltpu.sync_copy(vsz_hbm, vsz_v)
    vsz = vsz_v[:]
    bufs = [row_a, row_b]

    for k in range(ROWS_PER_W):
        row = base + k
        buf = bufs[k % 2]
        pltpu.sync_copy(vocab_hbm.at[row], buf)
        pltpu.sync_copy(feat_hbm.at[row], feat_v)
        pltpu.sync_copy(attn_hbm.at[row], attn_v)
        pltpu.sync_copy(src_hbm.at[row], src_v)

        def dot_body(i, a):
            return a + feat_v[pl.ds(i, L)] * w_v[pl.ds(i, L)]

        acc = plsc.parallel_loop(0, FEAT, L, carry=b_v[:])(dot_body)
        red_v[:] = plsc.cumsum(acc)
        x16 = plsc.load_gather(red_v, [jnp.full((L,), L - 1, jnp.int32)])
        pg = 1.0 / (1.0 + jnp.exp(-x16))
        one_m = 1.0 - pg

        def scale_body(i):
            buf[pl.ds(i, L)] = buf[pl.ds(i, L)] * pg

        plsc.parallel_loop(0, V, L, unroll=8)(scale_body)

        for c in range(S // L):
            idx = src_v[pl.ds(c * L, L)]
            vals = attn_v[pl.ds(c * L, L)] * one_m
            mask = idx < vsz
            safe = jnp.minimum(jnp.maximum(idx, 0), V - 1)
            plsc.addupdate_scatter(buf, [safe], vals, mask=mask)

        pltpu.sync_copy(buf, out_hbm.at[row])


def kernel(vocab_dist, attn_dist, context, state, emb, src_ids, vocab_size,
           W_c, W_s, W_y, b):
    feat = jnp.concatenate([context, state, emb], axis=1)
    w = jnp.concatenate([W_c[0], W_s[0], W_y[0]])
    b16 = jnp.pad(b.astype(jnp.float32), (0, L - 1))
    vsz16 = jnp.full((L,), vocab_size, dtype=jnp.int32)
    src = src_ids.astype(jnp.int32)
    return _pg_kernel(vocab_dist, feat, attn_dist, src, w, b16, vsz16)


# speedup vs baseline: 1.8464x; 1.1756x over previous
import functools

import jax
import jax.numpy as jnp
from jax import lax
from jax.experimental import pallas as pl
from jax.experimental.pallas import tpu as pltpu
from jax.experimental.pallas import tpu_sc as plsc

B = 128
V = 50000
S = 512
FEAT = 2560
NC = 2
NS = 16
L = 16
NW = NC * NS
ROWS_PER_W = B // NW

_MESH = plsc.VectorSubcoreMesh(
    core_axis_name="c", subcore_axis_name="s", num_cores=NC, num_subcores=NS
)


@functools.partial(
    pl.kernel,
    out_type=jax.ShapeDtypeStruct((B, V), jnp.float32),
    mesh=_MESH,
    compiler_params=pltpu.CompilerParams(needs_layout_passes=False),
    scratch_types=[
        pltpu.VMEM((V,), jnp.float32),
        pltpu.VMEM((V,), jnp.float32),
        pltpu.VMEM((ROWS_PER_W, FEAT), jnp.float32),
        pltpu.VMEM((ROWS_PER_W, S), jnp.float32),
        pltpu.VMEM((ROWS_PER_W, S), jnp.int32),
        pltpu.VMEM((FEAT,), jnp.float32),
        pltpu.VMEM((L,), jnp.float32),
        pltpu.VMEM((L,), jnp.int32),
        pltpu.VMEM((L,), jnp.float32),
        pltpu.SemaphoreType.DMA,
        pltpu.SemaphoreType.DMA,
        pltpu.SemaphoreType.DMA,
        pltpu.SemaphoreType.DMA,
    ],
)
def _pg_kernel(vocab_hbm, feat_hbm, attn_hbm, src_hbm, w_hbm, b_hbm, vsz_hbm,
               out_hbm, row_a, row_b, feat_v, attn_v, src_v, w_v, b_v, vsz_v,
               red_v, sem_in_a, sem_in_b, sem_out_a, sem_out_b):
    wid = lax.axis_index("s") * NC + lax.axis_index("c")
    base = wid * ROWS_PER_W
    bufs = [row_a, row_b]
    sems_in = [sem_in_a, sem_in_b]
    sems_out = [sem_out_a, sem_out_b]

    in_desc = [None] * ROWS_PER_W
    out_desc = [None] * ROWS_PER_W
    in_desc[0] = pltpu.async_copy(vocab_hbm.at[base], row_a, sem_in_a)

    pltpu.sync_copy(w_hbm, w_v)
    pltpu.sync_copy(b_hbm, b_v)
    pltpu.sync_copy(vsz_hbm, vsz_v)
    pltpu.sync_copy(feat_hbm.at[pl.ds(base, ROWS_PER_W)], feat_v)
    pltpu.sync_copy(attn_hbm.at[pl.ds(base, ROWS_PER_W)], attn_v)
    pltpu.sync_copy(src_hbm.at[pl.ds(base, ROWS_PER_W)], src_v)
    vsz = vsz_v[:]

    for k in range(ROWS_PER_W):
        buf = bufs[k % 2]

        def dot_body(i, a):
            return a + feat_v[k, pl.ds(i, L)] * w_v[pl.ds(i, L)]

        acc = plsc.parallel_loop(0, FEAT, L, carry=b_v[:])(dot_body)
        red_v[:] = plsc.cumsum(acc)
        x16 = plsc.load_gather(red_v, [jnp.full((L,), L - 1, jnp.int32)])
        pg = 1.0 / (1.0 + jnp.exp(-x16))
        one_m = 1.0 - pg

        if k + 1 < ROWS_PER_W:
            if k >= 1:
                out_desc[k - 1].wait()
            in_desc[k + 1] = pltpu.async_copy(
                vocab_hbm.at[base + k + 1], bufs[(k + 1) % 2],
                sems_in[(k + 1) % 2])

        in_desc[k].wait()

        def scale_body(i):
            buf[pl.ds(i, L)] = buf[pl.ds(i, L)] * pg

        plsc.parallel_loop(0, V, L, unroll=8)(scale_body)

        for c in range(S // L):
            idx = src_v[k, pl.ds(c * L, L)]
            vals = attn_v[k, pl.ds(c * L, L)] * one_m
            mask = idx < vsz
            safe = jnp.minimum(jnp.maximum(idx, 0), V - 1)
            plsc.addupdate_scatter(buf, [safe], vals, mask=mask)

        out_desc[k] = pltpu.async_copy(buf, out_hbm.at[base + k],
                                       sems_out[k % 2])

    out_desc[ROWS_PER_W - 2].wait()
    out_desc[ROWS_PER_W - 1].wait()


def kernel(vocab_dist, attn_dist, context, state, emb, src_ids, vocab_size,
           W_c, W_s, W_y, b):
    feat = jnp.concatenate([context, state, emb], axis=1)
    w = jnp.concatenate([W_c[0], W_s[0], W_y[0]])
    b16 = jnp.pad(b.astype(jnp.float32), (0, L - 1))
    vsz16 = jnp.full((L,), vocab_size, dtype=jnp.int32)
    src = src_ids.astype(jnp.int32)
    return _pg_kernel(vocab_dist, feat, attn_dist, src, w, b16, vsz16)
